# baseline (speedup 1.0000x reference)
import functools
import math

import jax
import jax.numpy as jnp
from jax import lax
from jax.experimental import pallas as pl
from jax.experimental.pallas import tpu as pltpu
from jax.experimental.pallas import tpu_sc as plsc

B, S, D = 1, 2048, 1024
H, KV, HD = 16, 4, 64
E, HID = 8, 2048
EPS = 1e-6

SB = 256
AQ = 512
AKV = 512
NQ = S // AQ
TILE = 256
PADDED = S + E * TILE
NTILES = PADDED // TILE
SCW = 128
QC = D // 4

f32 = jnp.float32
bf16 = jnp.bfloat16


def _rms(x, w):
    return w * x * lax.rsqrt(jnp.mean(x * x, axis=-1, keepdims=True) + EPS)


def _rope(x, c, s):
    half = x.shape[-1] // 2
    x1 = x[:, :half]
    x2 = x[:, half:]
    return jnp.concatenate([x1 * c - x2 * s, x1 * s + x2 * c], axis=1)


def _preattn_body(x_ref, ln1_ref, wq_ref, bq_ref, wk_ref, bk_ref, wv_ref, bv_ref,
                  cos_ref, sin_ref, qn_ref, kn_ref, q_ref, k_ref, v_ref):
    xb = x_ref[...]
    h = _rms(xb, ln1_ref[...]).astype(bf16)
    qall = jnp.dot(h, wq_ref[...], preferred_element_type=f32) + bq_ref[...]
    kall = jnp.dot(h, wk_ref[...], preferred_element_type=f32) + bk_ref[...]
    vall = jnp.dot(h, wv_ref[...], preferred_element_type=f32) + bv_ref[...]
    c = cos_ref[...]
    s = sin_ref[...]
    for head in range(H):
        qh = qall[:, head * HD:(head + 1) * HD]
        q_ref[head] = _rms(_rope(qh, c, s), qn_ref[...]).astype(bf16)
    for g in range(KV):
        kh = kall[:, g * HD:(g + 1) * HD]
        k_ref[g] = _rms(_rope(kh, c, s), kn_ref[...]).astype(bf16)
        v_ref[g] = vall[:, g * HD:(g + 1) * HD].astype(bf16)


def _preattn(x2d, ln1_w, wq, bq, wk, bk, wv, bv, cos, sin, qn_w, kn_w):
    nsb = S // SB
    return pl.pallas_call(
        _preattn_body,
        grid=(nsb,),
        in_specs=[
            pl.BlockSpec((SB, D), lambda i: (i, 0)),
            pl.BlockSpec((1, D), lambda i: (0, 0)),
            pl.BlockSpec((D, D), lambda i: (0, 0)),
            pl.BlockSpec((1, D), lambda i: (0, 0)),
            pl.BlockSpec((D, KV * HD), lambda i: (0, 0)),
            pl.BlockSpec((1, KV * HD), lambda i: (0, 0)),
            pl.BlockSpec((D, KV * HD), lambda i: (0, 0)),
            pl.BlockSpec((1, KV * HD), lambda i: (0, 0)),
            pl.BlockSpec((SB, HD // 2), lambda i: (i, 0)),
            pl.BlockSpec((SB, HD // 2), lambda i: (i, 0)),
            pl.BlockSpec((1, HD), lambda i: (0, 0)),
            pl.BlockSpec((1, HD), lambda i: (0, 0)),
        ],
        out_specs=[
            pl.BlockSpec((H, SB, HD), lambda i: (0, i, 0)),
            pl.BlockSpec((KV, SB, HD), lambda i: (0, i, 0)),
            pl.BlockSpec((KV, SB, HD), lambda i: (0, i, 0)),
        ],
        out_shape=[
            jax.ShapeDtypeStruct((H, S, HD), bf16),
            jax.ShapeDtypeStruct((KV, S, HD), bf16),
            jax.ShapeDtypeStruct((KV, S, HD), bf16),
        ],
    )(x2d, ln1_w, wq, bq, wk, bk, wv, bv, cos, sin, qn_w, kn_w)


def _attn_body(q_ref, k_ref, v_ref, wg_ref, bg_ref, neg_ref, o_ref, p_ref):
    sq = pl.program_id(1)
    q = q_ref[0]
    scale = 1.0 / math.sqrt(HD)

    def _scores(j):
        kc = k_ref[0, pl.ds(j * AKV, AKV), :]
        sc = lax.dot_general(q, kc, (((1,), (1,)), ((), ())),
                             preferred_element_type=f32) * scale
        return lax.cond(j == sq, lambda s: s + neg_ref[...], lambda s: s, sc)

    def pass1(j, m):
        return jnp.maximum(m, jnp.max(_scores(j), axis=1, keepdims=True))

    m = lax.fori_loop(0, sq + 1, pass1, jnp.full((AQ, 1), -1e30, f32))

    def pass2(j, l):
        p = jnp.exp(_scores(j) - m)
        p_ref[j] = p
        return l + jnp.sum(p, axis=1, keepdims=True)

    l = lax.fori_loop(0, sq + 1, pass2, jnp.zeros((AQ, 1), f32))
    inv_l = 1.0 / l

    def pass3(j, acc):
        p = (p_ref[j] * inv_l).astype(bf16)
        vc = v_ref[0, pl.ds(j * AKV, AKV), :]
        return acc + lax.dot_general(p, vc, (((1,), (0,)), ((), ())),
                                     preferred_element_type=f32)

    out = lax.fori_loop(0, sq + 1, pass3, jnp.zeros((AQ, HD), f32))
    gate = jax.nn.sigmoid(jnp.dot(out.astype(bf16), wg_ref[...],
                                  preferred_element_type=f32) + bg_ref[...])
    o_ref[0] = (out * gate).astype(bf16)


def _attention(q, k, v, wg, bg, neg):
    rep = H // KV
    return pl.pallas_call(
        _attn_body,
        grid=(H, NQ),
        in_specs=[
            pl.BlockSpec((1, AQ, HD), lambda h, sq: (h, sq, 0)),
            pl.BlockSpec((1, S, HD), lambda h, sq: (h // rep, 0, 0)),
            pl.BlockSpec((1, S, HD), lambda h, sq: (h // rep, 0, 0)),
            pl.BlockSpec((HD, HD), lambda h, sq: (0, 0)),
            pl.BlockSpec((1, HD), lambda h, sq: (0, 0)),
            pl.BlockSpec((AQ, AKV), lambda h, sq: (0, 0)),
        ],
        out_specs=pl.BlockSpec((1, AQ, HD), lambda h, sq: (h, sq, 0)),
        out_shape=jax.ShapeDtypeStruct((H, S, HD), bf16),
        scratch_shapes=[pltpu.VMEM((NQ, AQ, AKV), f32)],
    )(q, k, v, wg, bg, neg)


def _post_body(attn_ref, x_ref, wo_ref, bo_ref, ln2_ref, wr_ref, br_ref,
               x2_ref, oneh_ref, *h2q_refs):
    af = jnp.concatenate([attn_ref[head] for head in range(H)], axis=1)
    acc = jnp.dot(af, wo_ref[...], preferred_element_type=f32)
    x2 = x_ref[...] + acc + bo_ref[...]
    x2_ref[...] = x2
    h2 = _rms(x2, ln2_ref[...])
    for c in range(4):
        h2q_refs[c][...] = h2[:, c * QC:(c + 1) * QC]
    logits = jnp.dot(h2.astype(bf16), wr_ref[...],
                     preferred_element_type=f32) + br_ref[...]
    m = jnp.max(logits, axis=1, keepdims=True)
    io = lax.broadcasted_iota(jnp.int32, (SB, E), 1)
    cand = jnp.where(logits == m, io, E)
    t1 = jnp.min(cand, axis=1, keepdims=True)
    oneh_ref[...] = (io == t1).astype(bf16)


def _post_router(attn, x2d, wo, bo, ln2_w, wr, br):
    nsb = S // SB
    return pl.pallas_call(
        _post_body,
        grid=(nsb,),
        in_specs=[
            pl.BlockSpec((H, SB, HD), lambda i: (0, i, 0)),
            pl.BlockSpec((SB, D), lambda i: (i, 0)),
            pl.BlockSpec((D, D), lambda i: (0, 0)),
            pl.BlockSpec((1, D), lambda i: (0, 0)),
            pl.BlockSpec((1, D), lambda i: (0, 0)),
            pl.BlockSpec((D, E), lambda i: (0, 0)),
            pl.BlockSpec((1, E), lambda i: (0, 0)),
        ],
        out_specs=[
            pl.BlockSpec((SB, D), lambda i: (i, 0)),
            pl.BlockSpec((SB, E), lambda i: (i, 0)),
        ] + [pl.BlockSpec((SB, QC), lambda i: (i, 0)) for _ in range(4)],
        out_shape=[
            jax.ShapeDtypeStruct((S, D), f32),
            jax.ShapeDtypeStruct((S, E), bf16),
        ] + [jax.ShapeDtypeStruct((S, QC), f32) for _ in range(4)],
    )(attn, x2d, wo, bo, ln2_w, wr, br)


def _route_body(oneh_ref, dest_ref, gid_ref, valid_ref):
    oh = oneh_ref[...]
    ri = lax.broadcasted_iota(jnp.int32, (S, S), 0)
    ci = lax.broadcasted_iota(jnp.int32, (S, S), 1)
    mask = (ri >= ci).astype(bf16)
    ranks = jnp.dot(mask, oh, preferred_element_type=f32)
    counts = ranks[S - 1:S, :]
    pc = jnp.floor((counts + (TILE - 1)) * (1.0 / TILE)) * TILE
    eu = (lax.broadcasted_iota(jnp.int32, (E, E), 0) <
          lax.broadcasted_iota(jnp.int32, (E, E), 1)).astype(f32)
    pad_off = jnp.dot(pc, eu, preferred_element_type=f32)
    oh32 = oh.astype(f32)
    off_sel = jnp.sum(oh32 * pad_off, axis=1, keepdims=True)
    rank_sel = jnp.sum(oh32 * ranks, axis=1, keepdims=True)
    dest_ref[...] = (off_sel + rank_sel - 1.0).astype(jnp.int32)
    pad_end = pad_off + pc
    ts = (lax.broadcasted_iota(jnp.int32, (NTILES, E), 0) * TILE).astype(f32)
    g = jnp.minimum(jnp.sum((ts >= pad_end).astype(f32), axis=1, keepdims=True),
                    E - 1.0)
    gid_ref[...] = g.astype(jnp.int32)
    ioe = lax.broadcasted_iota(jnp.int32, (NTILES, E), 1).astype(f32)
    ohg = (ioe == g).astype(f32)
    end_sel = jnp.sum(ohg * (pad_off + counts), axis=1, keepdims=True)
    tstart = (lax.broadcasted_iota(jnp.int32, (NTILES, 1), 0) * TILE).astype(f32)
    valid_ref[...] = (tstart < end_sel).astype(jnp.int32)


def _routing(oneh):
    return pl.pallas_call(
        _route_body,
        out_shape=[
            jax.ShapeDtypeStruct((S, 1), jnp.int32),
            jax.ShapeDtypeStruct((NTILES, 1), jnp.int32),
            jax.ShapeDtypeStruct((NTILES, 1), jnp.int32),
        ],
    )(oneh)


def _sc_mesh():
    return plsc.VectorSubcoreMesh(core_axis_name="core", subcore_axis_name="subcore")


def _sc_scatter_rows(src, idx):
    w = src.shape[1]

    @functools.partial(
        pl.kernel,
        out_type=jax.ShapeDtypeStruct((PADDED, w), src.dtype),
        mesh=_sc_mesh(),
        scratch_types=[],
    )
    def run(src_hbm, idx_hbm, o_hbm):
        def body(x_vmem, i_vmem):
            pltpu.sync_copy(x_vmem, o_hbm.at[i_vmem.at[0]])

        pltpu.emit_pipeline(
            body,
            grid=(S // SCW,),
            in_specs=[
                pl.BlockSpec((SCW, w), lambda i: (i, 0)),
                pl.BlockSpec((1, SCW), lambda i: (0, i)),
            ],
            out_specs=[],
            core_axis_name=("core", "subcore"),
            dimension_semantics=(pltpu.PARALLEL,),
        )(src_hbm, idx_hbm)

    return run(src, idx)


def _sc_gather_rows(table, idx):
    w = table.shape[1]

    @functools.partial(
        pl.kernel,
        out_type=jax.ShapeDtypeStruct((S, w), table.dtype),
        mesh=_sc_mesh(),
        scratch_types=[],
    )
    def run(t_hbm, idx_hbm, o_hbm):
        def body(i_vmem, o_vmem):
            pltpu.sync_copy(t_hbm.at[i_vmem.at[0]], o_vmem)

        pltpu.emit_pipeline(
            body,
            grid=(S // SCW,),
            in_specs=[pl.BlockSpec((1, SCW), lambda i: (0, i))],
            out_specs=[pl.BlockSpec((SCW, w), lambda i: (i, 0))],
            core_axis_name=("core", "subcore"),
            dimension_semantics=(pltpu.PARALLEL,),
        )(idx_hbm, o_hbm)

    return run(table, idx)


def _moe_body(gid_ref, valid_ref, h0_ref, h1_ref, h2_ref, h3_ref,
              w1_ref, b1_ref, w2_ref, b2_ref, *yq_refs):
    t = pl.program_id(0)

    @pl.when(valid_ref[t] == 1)
    def _():
        hs = jnp.concatenate(
            [h0_ref[...], h1_ref[...], h2_ref[...], h3_ref[...]], axis=1)
        hh = jnp.dot(hs, w1_ref[0], preferred_element_type=f32) + b1_ref[0]
        a = hh[:, :HID]
        g = hh[:, HID:]
        act = (a * jax.nn.sigmoid(g)).astype(bf16)
        y = jnp.dot(act, w2_ref[0], preferred_element_type=f32) + b2_ref[0]
        for c in range(4):
            yq_refs[c][...] = y[:, c * QC:(c + 1) * QC]


def _moe(gid, valid, hq, w1b, b1, w2b, b2):
    grid_spec = pltpu.PrefetchScalarGridSpec(
        num_scalar_prefetch=2,
        grid=(NTILES,),
        in_specs=[
            pl.BlockSpec((TILE, QC), lambda t, g, vv: (t, 0)),
            pl.BlockSpec((TILE, QC), lambda t, g, vv: (t, 0)),
            pl.BlockSpec((TILE, QC), lambda t, g, vv: (t, 0)),
            pl.BlockSpec((TILE, QC), lambda t, g, vv: (t, 0)),
            pl.BlockSpec((1, D, 2 * HID), lambda t, g, vv: (g[t], 0, 0)),
            pl.BlockSpec((1, 1, 2 * HID), lambda t, g, vv: (g[t], 0, 0)),
            pl.BlockSpec((1, HID, D), lambda t, g, vv: (g[t], 0, 0)),
            pl.BlockSpec((1, 1, D), lambda t, g, vv: (g[t], 0, 0)),
        ],
        out_specs=[pl.BlockSpec((TILE, QC), lambda t, g, vv: (t, 0))
                   for _ in range(4)],
    )
    return pl.pallas_call(
        _moe_body,
        grid_spec=grid_spec,
        out_shape=[jax.ShapeDtypeStruct((PADDED, QC), f32) for _ in range(4)],
    )(gid, valid, hq[0], hq[1], hq[2], hq[3], w1b, b1, w2b, b2)


def _addres_body(a_ref, m0_ref, m1_ref, m2_ref, m3_ref, o_ref):
    a = a_ref[...]
    mrefs = (m0_ref, m1_ref, m2_ref, m3_ref)
    for c in range(4):
        o_ref[:, c * QC:(c + 1) * QC] = a[:, c * QC:(c + 1) * QC] + mrefs[c][...]


def _add_residual(a, mq):
    nsb = S // SB
    return pl.pallas_call(
        _addres_body,
        grid=(nsb,),
        in_specs=[pl.BlockSpec((SB, D), lambda i: (i, 0))] +
                 [pl.BlockSpec((SB, QC), lambda i: (i, 0)) for _ in range(4)],
        out_specs=pl.BlockSpec((SB, D), lambda i: (i, 0)),
        out_shape=jax.ShapeDtypeStruct((S, D), f32),
    )(a, mq[0], mq[1], mq[2], mq[3])


def kernel(x, ln1_w, Wq, bq, Wk, bk, Wv, bv, Wg, bg, Wo, bo, qn_w, kn_w,
           ln2_w, Wr, br, W1, b1, W2, b2, cos, sin):
    x2d = x.reshape(S, D)

    q, k, v = _preattn(x2d, ln1_w.reshape(1, D), Wq.astype(bf16),
                       bq.reshape(1, D), Wk.astype(bf16), bk.reshape(1, KV * HD),
                       Wv.astype(bf16), bv.reshape(1, KV * HD),
                       cos, sin, qn_w.reshape(1, HD), kn_w.reshape(1, HD))
    neg = jnp.where(lax.broadcasted_iota(jnp.int32, (AQ, AKV), 0) >=
                    lax.broadcasted_iota(jnp.int32, (AQ, AKV), 1),
                    0.0, -1e30).astype(f32)
    attn = _attention(q, k, v, Wg.astype(bf16), bg.reshape(1, HD), neg)
    x2, oneh, *h2q = _post_router(attn, x2d, Wo.astype(bf16), bo.reshape(1, D),
                                  ln2_w.reshape(1, D), Wr.astype(bf16),
                                  br.reshape(1, E))
    dest, gid, valid = _routing(oneh)
    dest_row = dest.reshape(1, S)
    hs_q = [_sc_scatter_rows(h2q[c], dest_row) for c in range(4)]
    yq = _moe(gid.reshape(NTILES), valid.reshape(NTILES), hs_q,
              W1, b1.reshape(E, 1, 2 * HID),
              W2.astype(bf16), b2.reshape(E, 1, D))
    mq = [_sc_gather_rows(yq[c], dest_row) for c in range(4)]
    out = _add_residual(x2, mq)
    return out.reshape(B, S, D)


# speedup vs baseline: 1.3334x; 1.3334x over previous
import functools
import math

import jax
import jax.numpy as jnp
from jax import lax
from jax.experimental import pallas as pl
from jax.experimental.pallas import tpu as pltpu
from jax.experimental.pallas import tpu_sc as plsc

B, S, D = 1, 2048, 1024
H, KV, HD = 16, 4, 64
E, HID = 8, 2048
EPS = 1e-6

SB = 256
AQ = 512
AKV = 512
NQ = S // AQ
TILE = 256
PADDED = S + E * TILE
NTILES = PADDED // TILE
SCW = 128
QC = D // 4

f32 = jnp.float32
bf16 = jnp.bfloat16


def _rms(x, w):
    return w * x * lax.rsqrt(jnp.mean(x * x, axis=-1, keepdims=True) + EPS)


def _rope(x, c, s):
    half = x.shape[-1] // 2
    x1 = x[:, :half]
    x2 = x[:, half:]
    return jnp.concatenate([x1 * c - x2 * s, x1 * s + x2 * c], axis=1)


def _preattn_body(x_ref, ln1_ref, wq_ref, bq_ref, wk_ref, bk_ref, wv_ref, bv_ref,
                  cos_ref, sin_ref, qn_ref, kn_ref, q_ref, k_ref, v_ref):
    xb = x_ref[...]
    h = _rms(xb, ln1_ref[...]).astype(bf16)
    qall = jnp.dot(h, wq_ref[...], preferred_element_type=f32) + bq_ref[...]
    kall = jnp.dot(h, wk_ref[...], preferred_element_type=f32) + bk_ref[...]
    vall = jnp.dot(h, wv_ref[...], preferred_element_type=f32) + bv_ref[...]
    c = cos_ref[...]
    s = sin_ref[...]
    for head in range(H):
        qh = qall[:, head * HD:(head + 1) * HD]
        q_ref[head] = _rms(_rope(qh, c, s), qn_ref[...]).astype(bf16)
    for g in range(KV):
        kh = kall[:, g * HD:(g + 1) * HD]
        k_ref[g] = _rms(_rope(kh, c, s), kn_ref[...]).astype(bf16)
        v_ref[g] = vall[:, g * HD:(g + 1) * HD].astype(bf16)


def _preattn(x2d, ln1_w, wq, bq, wk, bk, wv, bv, cos, sin, qn_w, kn_w):
    nsb = S // SB
    return pl.pallas_call(
        _preattn_body,
        grid=(nsb,),
        in_specs=[
            pl.BlockSpec((SB, D), lambda i: (i, 0)),
            pl.BlockSpec((1, D), lambda i: (0, 0)),
            pl.BlockSpec((D, D), lambda i: (0, 0)),
            pl.BlockSpec((1, D), lambda i: (0, 0)),
            pl.BlockSpec((D, KV * HD), lambda i: (0, 0)),
            pl.BlockSpec((1, KV * HD), lambda i: (0, 0)),
            pl.BlockSpec((D, KV * HD), lambda i: (0, 0)),
            pl.BlockSpec((1, KV * HD), lambda i: (0, 0)),
            pl.BlockSpec((SB, HD // 2), lambda i: (i, 0)),
            pl.BlockSpec((SB, HD // 2), lambda i: (i, 0)),
            pl.BlockSpec((1, HD), lambda i: (0, 0)),
            pl.BlockSpec((1, HD), lambda i: (0, 0)),
        ],
        out_specs=[
            pl.BlockSpec((H, SB, HD), lambda i: (0, i, 0)),
            pl.BlockSpec((KV, SB, HD), lambda i: (0, i, 0)),
            pl.BlockSpec((KV, SB, HD), lambda i: (0, i, 0)),
        ],
        out_shape=[
            jax.ShapeDtypeStruct((H, S, HD), bf16),
            jax.ShapeDtypeStruct((KV, S, HD), bf16),
            jax.ShapeDtypeStruct((KV, S, HD), bf16),
        ],
    )(x2d, ln1_w, wq, bq, wk, bk, wv, bv, cos, sin, qn_w, kn_w)


def _attn_body(q_ref, k_ref, v_ref, wg_ref, bg_ref, neg_ref, o_ref, p_ref):
    sq = pl.program_id(1)
    q = q_ref[0]
    scale = 1.0 / math.sqrt(HD)

    def pass1(j, m):
        kc = k_ref[0, pl.ds(j * AKV, AKV), :]
        sc = lax.dot_general(q, kc, (((1,), (1,)), ((), ())),
                             preferred_element_type=f32) * scale
        sc = jnp.where(j == sq, sc + neg_ref[...], sc)
        p_ref[j] = sc
        return jnp.maximum(m, jnp.max(sc, axis=1, keepdims=True))

    m = lax.fori_loop(0, sq + 1, pass1, jnp.full((AQ, 1), -1e30, f32))

    def pass2(j, l):
        p = jnp.exp(p_ref[j] - m)
        p_ref[j] = p
        return l + jnp.sum(p, axis=1, keepdims=True)

    l = lax.fori_loop(0, sq + 1, pass2, jnp.zeros((AQ, 1), f32))
    inv_l = 1.0 / l

    def pass3(j, acc):
        p = (p_ref[j] * inv_l).astype(bf16)
        vc = v_ref[0, pl.ds(j * AKV, AKV), :]
        return acc + lax.dot_general(p, vc, (((1,), (0,)), ((), ())),
                                     preferred_element_type=f32)

    out = lax.fori_loop(0, sq + 1, pass3, jnp.zeros((AQ, HD), f32))
    gate = jax.nn.sigmoid(jnp.dot(out.astype(bf16), wg_ref[...],
                                  preferred_element_type=f32) + bg_ref[...])
    o_ref[0] = (out * gate).astype(bf16)


def _attention(q, k, v, wg, bg, neg):
    rep = H // KV
    return pl.pallas_call(
        _attn_body,
        grid=(H, NQ),
        in_specs=[
            pl.BlockSpec((1, AQ, HD), lambda h, sq: (h, sq, 0)),
            pl.BlockSpec((1, S, HD), lambda h, sq: (h // rep, 0, 0)),
            pl.BlockSpec((1, S, HD), lambda h, sq: (h // rep, 0, 0)),
            pl.BlockSpec((HD, HD), lambda h, sq: (0, 0)),
            pl.BlockSpec((1, HD), lambda h, sq: (0, 0)),
            pl.BlockSpec((AQ, AKV), lambda h, sq: (0, 0)),
        ],
        out_specs=pl.BlockSpec((1, AQ, HD), lambda h, sq: (h, sq, 0)),
        out_shape=jax.ShapeDtypeStruct((H, S, HD), bf16),
        scratch_shapes=[pltpu.VMEM((NQ, AQ, AKV), f32)],
    )(q, k, v, wg, bg, neg)


def _post_body(attn_ref, x_ref, wo_ref, bo_ref, ln2_ref, wr_ref, br_ref,
               x2_ref, oneh_ref, *h2q_refs):
    af = jnp.concatenate([attn_ref[head] for head in range(H)], axis=1)
    acc = jnp.dot(af, wo_ref[...], preferred_element_type=f32)
    x2 = x_ref[...] + acc + bo_ref[...]
    x2_ref[...] = x2
    h2 = _rms(x2, ln2_ref[...])
    for c in range(4):
        h2q_refs[c][...] = h2[:, c * QC:(c + 1) * QC]
    logits = jnp.dot(h2.astype(bf16), wr_ref[...],
                     preferred_element_type=f32) + br_ref[...]
    m = jnp.max(logits, axis=1, keepdims=True)
    io = lax.broadcasted_iota(jnp.int32, (SB, E), 1)
    cand = jnp.where(logits == m, io, E)
    t1 = jnp.min(cand, axis=1, keepdims=True)
    oneh_ref[...] = (io == t1).astype(bf16)


def _post_router(attn, x2d, wo, bo, ln2_w, wr, br):
    nsb = S // SB
    return pl.pallas_call(
        _post_body,
        grid=(nsb,),
        in_specs=[
            pl.BlockSpec((H, SB, HD), lambda i: (0, i, 0)),
            pl.BlockSpec((SB, D), lambda i: (i, 0)),
            pl.BlockSpec((D, D), lambda i: (0, 0)),
            pl.BlockSpec((1, D), lambda i: (0, 0)),
            pl.BlockSpec((1, D), lambda i: (0, 0)),
            pl.BlockSpec((D, E), lambda i: (0, 0)),
            pl.BlockSpec((1, E), lambda i: (0, 0)),
        ],
        out_specs=[
            pl.BlockSpec((SB, D), lambda i: (i, 0)),
            pl.BlockSpec((SB, E), lambda i: (i, 0)),
        ] + [pl.BlockSpec((SB, QC), lambda i: (i, 0)) for _ in range(4)],
        out_shape=[
            jax.ShapeDtypeStruct((S, D), f32),
            jax.ShapeDtypeStruct((S, E), bf16),
        ] + [jax.ShapeDtypeStruct((S, QC), f32) for _ in range(4)],
    )(attn, x2d, wo, bo, ln2_w, wr, br)


def _route_body(oneh_ref, dest_ref, gid_ref, valid_ref):
    oh = oneh_ref[...]
    ri = lax.broadcasted_iota(jnp.int32, (S, S), 0)
    ci = lax.broadcasted_iota(jnp.int32, (S, S), 1)
    mask = (ri >= ci).astype(bf16)
    ranks = jnp.dot(mask, oh, preferred_element_type=f32)
    counts = ranks[S - 1:S, :]
    pc = jnp.floor((counts + (TILE - 1)) * (1.0 / TILE)) * TILE
    eu = (lax.broadcasted_iota(jnp.int32, (E, E), 0) <
          lax.broadcasted_iota(jnp.int32, (E, E), 1)).astype(f32)
    pad_off = jnp.dot(pc, eu, preferred_element_type=f32)
    oh32 = oh.astype(f32)
    off_sel = jnp.sum(oh32 * pad_off, axis=1, keepdims=True)
    rank_sel = jnp.sum(oh32 * ranks, axis=1, keepdims=True)
    dest_ref[...] = (off_sel + rank_sel - 1.0).astype(jnp.int32)
    pad_end = pad_off + pc
    ts = (lax.broadcasted_iota(jnp.int32, (NTILES, E), 0) * TILE).astype(f32)
    g = jnp.minimum(jnp.sum((ts >= pad_end).astype(f32), axis=1, keepdims=True),
                    E - 1.0)
    gid_ref[...] = g.astype(jnp.int32)
    ioe = lax.broadcasted_iota(jnp.int32, (NTILES, E), 1).astype(f32)
    ohg = (ioe == g).astype(f32)
    end_sel = jnp.sum(ohg * (pad_off + counts), axis=1, keepdims=True)
    tstart = (lax.broadcasted_iota(jnp.int32, (NTILES, 1), 0) * TILE).astype(f32)
    valid_ref[...] = (tstart < end_sel).astype(jnp.int32)


def _routing(oneh):
    return pl.pallas_call(
        _route_body,
        out_shape=[
            jax.ShapeDtypeStruct((S, 1), jnp.int32),
            jax.ShapeDtypeStruct((NTILES, 1), jnp.int32),
            jax.ShapeDtypeStruct((NTILES, 1), jnp.int32),
        ],
    )(oneh)


def _sc_mesh():
    return plsc.VectorSubcoreMesh(core_axis_name="core", subcore_axis_name="subcore")


def _sc_scatter_rows(src, idx):
    w = src.shape[1]

    @functools.partial(
        pl.kernel,
        out_type=jax.ShapeDtypeStruct((PADDED, w), src.dtype),
        mesh=_sc_mesh(),
        scratch_types=[],
    )
    def run(src_hbm, idx_hbm, o_hbm):
        def body(x_vmem, i_vmem):
            pltpu.sync_copy(x_vmem, o_hbm.at[i_vmem.at[0]])

        pltpu.emit_pipeline(
            body,
            grid=(S // SCW,),
            in_specs=[
                pl.BlockSpec((SCW, w), lambda i: (i, 0)),
                pl.BlockSpec((1, SCW), lambda i: (0, i)),
            ],
            out_specs=[],
            core_axis_name=("core", "subcore"),
            dimension_semantics=(pltpu.PARALLEL,),
        )(src_hbm, idx_hbm)

    return run(src, idx)


def _sc_gather_rows(table, idx):
    w = table.shape[1]

    @functools.partial(
        pl.kernel,
        out_type=jax.ShapeDtypeStruct((S, w), table.dtype),
        mesh=_sc_mesh(),
        scratch_types=[],
    )
    def run(t_hbm, idx_hbm, o_hbm):
        def body(i_vmem, o_vmem):
            pltpu.sync_copy(t_hbm.at[i_vmem.at[0]], o_vmem)

        pltpu.emit_pipeline(
            body,
            grid=(S // SCW,),
            in_specs=[pl.BlockSpec((1, SCW), lambda i: (0, i))],
            out_specs=[pl.BlockSpec((SCW, w), lambda i: (i, 0))],
            core_axis_name=("core", "subcore"),
            dimension_semantics=(pltpu.PARALLEL,),
        )(idx_hbm, o_hbm)

    return run(table, idx)


def _moe_body(gid_ref, valid_ref, h0_ref, h1_ref, h2_ref, h3_ref,
              w1_ref, b1_ref, w2_ref, b2_ref, *yq_refs):
    t = pl.program_id(0)

    @pl.when(valid_ref[t] == 1)
    def _():
        hs = jnp.concatenate(
            [h0_ref[...], h1_ref[...], h2_ref[...], h3_ref[...]], axis=1)
        hh = jnp.dot(hs, w1_ref[0], preferred_element_type=f32) + b1_ref[0]
        a = hh[:, :HID]
        g = hh[:, HID:]
        act = a * jax.nn.sigmoid(g)
        y = jnp.dot(act, w2_ref[0], preferred_element_type=f32) + b2_ref[0]
        for c in range(4):
            yq_refs[c][...] = y[:, c * QC:(c + 1) * QC]


def _moe(gid, valid, hq, w1b, b1, w2b, b2):
    grid_spec = pltpu.PrefetchScalarGridSpec(
        num_scalar_prefetch=2,
        grid=(NTILES,),
        in_specs=[
            pl.BlockSpec((TILE, QC), lambda t, g, vv: (t, 0)),
            pl.BlockSpec((TILE, QC), lambda t, g, vv: (t, 0)),
            pl.BlockSpec((TILE, QC), lambda t, g, vv: (t, 0)),
            pl.BlockSpec((TILE, QC), lambda t, g, vv: (t, 0)),
            pl.BlockSpec((1, D, 2 * HID), lambda t, g, vv: (g[t], 0, 0)),
            pl.BlockSpec((1, 1, 2 * HID), lambda t, g, vv: (g[t], 0, 0)),
            pl.BlockSpec((1, HID, D), lambda t, g, vv: (g[t], 0, 0)),
            pl.BlockSpec((1, 1, D), lambda t, g, vv: (g[t], 0, 0)),
        ],
        out_specs=[pl.BlockSpec((TILE, QC), lambda t, g, vv: (t, 0))
                   for _ in range(4)],
    )
    return pl.pallas_call(
        _moe_body,
        grid_spec=grid_spec,
        out_shape=[jax.ShapeDtypeStruct((PADDED, QC), f32) for _ in range(4)],
    )(gid, valid, hq[0], hq[1], hq[2], hq[3], w1b, b1, w2b, b2)


def _addres_body(a_ref, m0_ref, m1_ref, m2_ref, m3_ref, o_ref):
    a = a_ref[...]
    mrefs = (m0_ref, m1_ref, m2_ref, m3_ref)
    for c in range(4):
        o_ref[:, c * QC:(c + 1) * QC] = a[:, c * QC:(c + 1) * QC] + mrefs[c][...]


def _add_residual(a, mq):
    nsb = S // SB
    return pl.pallas_call(
        _addres_body,
        grid=(nsb,),
        in_specs=[pl.BlockSpec((SB, D), lambda i: (i, 0))] +
                 [pl.BlockSpec((SB, QC), lambda i: (i, 0)) for _ in range(4)],
        out_specs=pl.BlockSpec((SB, D), lambda i: (i, 0)),
        out_shape=jax.ShapeDtypeStruct((S, D), f32),
    )(a, mq[0], mq[1], mq[2], mq[3])


def kernel(x, ln1_w, Wq, bq, Wk, bk, Wv, bv, Wg, bg, Wo, bo, qn_w, kn_w,
           ln2_w, Wr, br, W1, b1, W2, b2, cos, sin):
    x2d = x.reshape(S, D)

    q, k, v = _preattn(x2d, ln1_w.reshape(1, D), Wq.astype(bf16),
                       bq.reshape(1, D), Wk.astype(bf16), bk.reshape(1, KV * HD),
                       Wv.astype(bf16), bv.reshape(1, KV * HD),
                       cos, sin, qn_w.reshape(1, HD), kn_w.reshape(1, HD))
    neg = jnp.where(lax.broadcasted_iota(jnp.int32, (AQ, AKV), 0) >=
                    lax.broadcasted_iota(jnp.int32, (AQ, AKV), 1),
                    0.0, -1e30).astype(f32)
    attn = _attention(q, k, v, Wg.astype(bf16), bg.reshape(1, HD), neg)
    x2, oneh, *h2q = _post_router(attn, x2d, Wo.astype(bf16), bo.reshape(1, D),
                                  ln2_w.reshape(1, D), Wr.astype(bf16),
                                  br.reshape(1, E))
    dest, gid, valid = _routing(oneh)
    dest_row = dest.reshape(1, S)
    hs_q = [_sc_scatter_rows(h2q[c], dest_row) for c in range(4)]
    yq = _moe(gid.reshape(NTILES), valid.reshape(NTILES), hs_q,
              W1, b1.reshape(E, 1, 2 * HID),
              W2, b2.reshape(E, 1, D))
    mq = [_sc_gather_rows(yq[c], dest_row) for c in range(4)]
    out = _add_residual(x2, mq)
    return out.reshape(B, S, D)


# speedup vs baseline: 1.3470x; 1.0103x over previous
import functools
import math

import jax
import jax.numpy as jnp
from jax import lax
from jax.experimental import pallas as pl
from jax.experimental.pallas import tpu as pltpu
from jax.experimental.pallas import tpu_sc as plsc

B, S, D = 1, 2048, 1024
H, KV, HD = 16, 4, 64
E, HID = 8, 2048
EPS = 1e-6

SB = 256
AQ = 1024
AKV = 512
NQ = S // AQ
NCH = S // AKV
TILE = 256
PADDED = S + E * TILE
NTILES = PADDED // TILE
SCW = 128
QC = D // 4

f32 = jnp.float32
bf16 = jnp.bfloat16


def _rms(x, w):
    return w * x * lax.rsqrt(jnp.mean(x * x, axis=-1, keepdims=True) + EPS)


def _rope(x, c, s):
    half = x.shape[-1] // 2
    x1 = x[:, :half]
    x2 = x[:, half:]
    return jnp.concatenate([x1 * c - x2 * s, x1 * s + x2 * c], axis=1)


def _preattn_body(x_ref, ln1_ref, wq_ref, bq_ref, wk_ref, bk_ref, wv_ref, bv_ref,
                  cos_ref, sin_ref, qn_ref, kn_ref, q_ref, k_ref, v_ref):
    xb = x_ref[...]
    h = _rms(xb, ln1_ref[...]).astype(bf16)
    qall = jnp.dot(h, wq_ref[...], preferred_element_type=f32) + bq_ref[...]
    kall = jnp.dot(h, wk_ref[...], preferred_element_type=f32) + bk_ref[...]
    vall = jnp.dot(h, wv_ref[...], preferred_element_type=f32) + bv_ref[...]
    c = cos_ref[...]
    s = sin_ref[...]
    for head in range(H):
        qh = qall[:, head * HD:(head + 1) * HD]
        q_ref[head] = _rms(_rope(qh, c, s), qn_ref[...]).astype(bf16)
    for g in range(KV):
        kh = kall[:, g * HD:(g + 1) * HD]
        k_ref[g] = _rms(_rope(kh, c, s), kn_ref[...]).astype(bf16)
        v_ref[g] = vall[:, g * HD:(g + 1) * HD].astype(bf16)


def _preattn(x2d, ln1_w, wq, bq, wk, bk, wv, bv, cos, sin, qn_w, kn_w):
    nsb = S // SB
    return pl.pallas_call(
        _preattn_body,
        grid=(nsb,),
        in_specs=[
            pl.BlockSpec((SB, D), lambda i: (i, 0)),
            pl.BlockSpec((1, D), lambda i: (0, 0)),
            pl.BlockSpec((D, D), lambda i: (0, 0)),
            pl.BlockSpec((1, D), lambda i: (0, 0)),
            pl.BlockSpec((D, KV * HD), lambda i: (0, 0)),
            pl.BlockSpec((1, KV * HD), lambda i: (0, 0)),
            pl.BlockSpec((D, KV * HD), lambda i: (0, 0)),
            pl.BlockSpec((1, KV * HD), lambda i: (0, 0)),
            pl.BlockSpec((SB, HD // 2), lambda i: (i, 0)),
            pl.BlockSpec((SB, HD // 2), lambda i: (i, 0)),
            pl.BlockSpec((1, HD), lambda i: (0, 0)),
            pl.BlockSpec((1, HD), lambda i: (0, 0)),
        ],
        out_specs=[
            pl.BlockSpec((H, SB, HD), lambda i: (0, i, 0)),
            pl.BlockSpec((KV, SB, HD), lambda i: (0, i, 0)),
            pl.BlockSpec((KV, SB, HD), lambda i: (0, i, 0)),
        ],
        out_shape=[
            jax.ShapeDtypeStruct((H, S, HD), bf16),
            jax.ShapeDtypeStruct((KV, S, HD), bf16),
            jax.ShapeDtypeStruct((KV, S, HD), bf16),
        ],
    )(x2d, ln1_w, wq, bq, wk, bk, wv, bv, cos, sin, qn_w, kn_w)


def _attn_body(q_ref, k_ref, v_ref, wg_ref, bg_ref, nega_ref, negb_ref,
               o_ref, p_ref):
    sq = pl.program_id(1)
    q = q_ref[0]
    scale = 1.0 / math.sqrt(HD)
    nch = 2 * sq + 2

    def pass1(j, m):
        kc = k_ref[0, pl.ds(j * AKV, AKV), :]
        sc = lax.dot_general(q, kc, (((1,), (1,)), ((), ())),
                             preferred_element_type=f32) * scale
        sc = jnp.where(j == 2 * sq, sc + nega_ref[...], sc)
        sc = jnp.where(j == 2 * sq + 1, sc + negb_ref[...], sc)
        p_ref[j] = sc
        return jnp.maximum(m, jnp.max(sc, axis=1, keepdims=True))

    m = lax.fori_loop(0, nch, pass1, jnp.full((AQ, 1), -1e30, f32))

    def pass2(j, l):
        p = jnp.exp(p_ref[j] - m)
        p_ref[j] = p
        return l + jnp.sum(p, axis=1, keepdims=True)

    l = lax.fori_loop(0, nch, pass2, jnp.zeros((AQ, 1), f32))
    inv_l = 1.0 / l

    def pass3(j, acc):
        p = (p_ref[j] * inv_l).astype(bf16)
        vc = v_ref[0, pl.ds(j * AKV, AKV), :]
        return acc + lax.dot_general(p, vc, (((1,), (0,)), ((), ())),
                                     preferred_element_type=f32)

    out = lax.fori_loop(0, nch, pass3, jnp.zeros((AQ, HD), f32))
    gate = jax.nn.sigmoid(jnp.dot(out.astype(bf16), wg_ref[...],
                                  preferred_element_type=f32) + bg_ref[...])
    o_ref[0] = (out * gate).astype(bf16)


def _attention(q, k, v, wg, bg, nega, negb):
    rep = H // KV
    return pl.pallas_call(
        _attn_body,
        grid=(H, NQ),
        in_specs=[
            pl.BlockSpec((1, AQ, HD), lambda h, sq: (h, sq, 0)),
            pl.BlockSpec((1, S, HD), lambda h, sq: (h // rep, 0, 0)),
            pl.BlockSpec((1, S, HD), lambda h, sq: (h // rep, 0, 0)),
            pl.BlockSpec((HD, HD), lambda h, sq: (0, 0)),
            pl.BlockSpec((1, HD), lambda h, sq: (0, 0)),
            pl.BlockSpec((AQ, AKV), lambda h, sq: (0, 0)),
            pl.BlockSpec((AQ, AKV), lambda h, sq: (0, 0)),
        ],
        out_specs=pl.BlockSpec((1, AQ, HD), lambda h, sq: (h, sq, 0)),
        out_shape=jax.ShapeDtypeStruct((H, S, HD), bf16),
        scratch_shapes=[pltpu.VMEM((NCH, AQ, AKV), f32)],
    )(q, k, v, wg, bg, nega, negb)


def _post_body(attn_ref, x_ref, wo_ref, bo_ref, ln2_ref, wr_ref, br_ref,
               x2_ref, oneh_ref, *h2q_refs):
    af = jnp.concatenate([attn_ref[head] for head in range(H)], axis=1)
    acc = jnp.dot(af, wo_ref[...], preferred_element_type=f32)
    x2 = x_ref[...] + acc + bo_ref[...]
    x2_ref[...] = x2
    h2 = _rms(x2, ln2_ref[...])
    for c in range(4):
        h2q_refs[c][...] = h2[:, c * QC:(c + 1) * QC]
    logits = jnp.dot(h2.astype(bf16), wr_ref[...],
                     preferred_element_type=f32) + br_ref[...]
    m = jnp.max(logits, axis=1, keepdims=True)
    io = lax.broadcasted_iota(jnp.int32, (SB, E), 1)
    cand = jnp.where(logits == m, io, E)
    t1 = jnp.min(cand, axis=1, keepdims=True)
    oneh_ref[...] = (io == t1).astype(bf16)


def _post_router(attn, x2d, wo, bo, ln2_w, wr, br):
    nsb = S // SB
    return pl.pallas_call(
        _post_body,
        grid=(nsb,),
        in_specs=[
            pl.BlockSpec((H, SB, HD), lambda i: (0, i, 0)),
            pl.BlockSpec((SB, D), lambda i: (i, 0)),
            pl.BlockSpec((D, D), lambda i: (0, 0)),
            pl.BlockSpec((1, D), lambda i: (0, 0)),
            pl.BlockSpec((1, D), lambda i: (0, 0)),
            pl.BlockSpec((D, E), lambda i: (0, 0)),
            pl.BlockSpec((1, E), lambda i: (0, 0)),
        ],
        out_specs=[
            pl.BlockSpec((SB, D), lambda i: (i, 0)),
            pl.BlockSpec((SB, E), lambda i: (i, 0)),
        ] + [pl.BlockSpec((SB, QC), lambda i: (i, 0)) for _ in range(4)],
        out_shape=[
            jax.ShapeDtypeStruct((S, D), f32),
            jax.ShapeDtypeStruct((S, E), bf16),
        ] + [jax.ShapeDtypeStruct((S, QC), f32) for _ in range(4)],
    )(attn, x2d, wo, bo, ln2_w, wr, br)


def _route_body(oneh_ref, dest_ref, gid_ref, valid_ref):
    oh = oneh_ref[...]
    ri = lax.broadcasted_iota(jnp.int32, (S, S), 0)
    ci = lax.broadcasted_iota(jnp.int32, (S, S), 1)
    mask = (ri >= ci).astype(bf16)
    ranks = jnp.dot(mask, oh, preferred_element_type=f32)
    counts = ranks[S - 1:S, :]
    pc = jnp.floor((counts + (TILE - 1)) * (1.0 / TILE)) * TILE
    eu = (lax.broadcasted_iota(jnp.int32, (E, E), 0) <
          lax.broadcasted_iota(jnp.int32, (E, E), 1)).astype(f32)
    pad_off = jnp.dot(pc, eu, preferred_element_type=f32)
    oh32 = oh.astype(f32)
    off_sel = jnp.sum(oh32 * pad_off, axis=1, keepdims=True)
    rank_sel = jnp.sum(oh32 * ranks, axis=1, keepdims=True)
    dest_ref[...] = (off_sel + rank_sel - 1.0).astype(jnp.int32)
    pad_end = pad_off + pc
    ts = (lax.broadcasted_iota(jnp.int32, (NTILES, E), 0) * TILE).astype(f32)
    g = jnp.minimum(jnp.sum((ts >= pad_end).astype(f32), axis=1, keepdims=True),
                    E - 1.0)
    gid_ref[...] = g.astype(jnp.int32)
    ioe = lax.broadcasted_iota(jnp.int32, (NTILES, E), 1).astype(f32)
    ohg = (ioe == g).astype(f32)
    end_sel = jnp.sum(ohg * (pad_off + counts), axis=1, keepdims=True)
    tstart = (lax.broadcasted_iota(jnp.int32, (NTILES, 1), 0) * TILE).astype(f32)
    valid_ref[...] = (tstart < end_sel).astype(jnp.int32)


def _routing(oneh):
    return pl.pallas_call(
        _route_body,
        out_shape=[
            jax.ShapeDtypeStruct((S, 1), jnp.int32),
            jax.ShapeDtypeStruct((NTILES, 1), jnp.int32),
            jax.ShapeDtypeStruct((NTILES, 1), jnp.int32),
        ],
    )(oneh)


def _sc_mesh():
    return plsc.VectorSubcoreMesh(core_axis_name="core", subcore_axis_name="subcore")


def _sc_scatter_rows(src, idx):
    w = src.shape[1]

    @functools.partial(
        pl.kernel,
        out_type=jax.ShapeDtypeStruct((PADDED, w), src.dtype),
        mesh=_sc_mesh(),
        scratch_types=[],
    )
    def run(src_hbm, idx_hbm, o_hbm):
        def body(x_vmem, i_vmem):
            pltpu.sync_copy(x_vmem, o_hbm.at[i_vmem.at[0]])

        pltpu.emit_pipeline(
            body,
            grid=(S // SCW,),
            in_specs=[
                pl.BlockSpec((SCW, w), lambda i: (i, 0)),
                pl.BlockSpec((1, SCW), lambda i: (0, i)),
            ],
            out_specs=[],
            core_axis_name=("core", "subcore"),
            dimension_semantics=(pltpu.PARALLEL,),
        )(src_hbm, idx_hbm)

    return run(src, idx)


def _sc_gather_rows(table, idx):
    w = table.shape[1]

    @functools.partial(
        pl.kernel,
        out_type=jax.ShapeDtypeStruct((S, w), table.dtype),
        mesh=_sc_mesh(),
        scratch_types=[],
    )
    def run(t_hbm, idx_hbm, o_hbm):
        def body(i_vmem, o_vmem):
            pltpu.sync_copy(t_hbm.at[i_vmem.at[0]], o_vmem)

        pltpu.emit_pipeline(
            body,
            grid=(S // SCW,),
            in_specs=[pl.BlockSpec((1, SCW), lambda i: (0, i))],
            out_specs=[pl.BlockSpec((SCW, w), lambda i: (i, 0))],
            core_axis_name=("core", "subcore"),
            dimension_semantics=(pltpu.PARALLEL,),
        )(idx_hbm, o_hbm)

    return run(table, idx)


def _moe_body(gid_ref, valid_ref, h0_ref, h1_ref, h2_ref, h3_ref,
              w1_ref, b1_ref, w2_ref, b2_ref, *yq_refs):
    t = pl.program_id(0)

    @pl.when(valid_ref[t] == 1)
    def _():
        hs = jnp.concatenate(
            [h0_ref[...], h1_ref[...], h2_ref[...], h3_ref[...]], axis=1)
        hh = jnp.dot(hs, w1_ref[0], preferred_element_type=f32) + b1_ref[0]
        a = hh[:, :HID]
        g = hh[:, HID:]
        act = a * jax.nn.sigmoid(g)
        y = jnp.dot(act, w2_ref[0], preferred_element_type=f32) + b2_ref[0]
        for c in range(4):
            yq_refs[c][...] = y[:, c * QC:(c + 1) * QC]


def _moe(gid, valid, hq, w1b, b1, w2b, b2):
    grid_spec = pltpu.PrefetchScalarGridSpec(
        num_scalar_prefetch=2,
        grid=(NTILES,),
        in_specs=[
            pl.BlockSpec((TILE, QC), lambda t, g, vv: (t, 0)),
            pl.BlockSpec((TILE, QC), lambda t, g, vv: (t, 0)),
            pl.BlockSpec((TILE, QC), lambda t, g, vv: (t, 0)),
            pl.BlockSpec((TILE, QC), lambda t, g, vv: (t, 0)),
            pl.BlockSpec((1, D, 2 * HID), lambda t, g, vv: (g[t], 0, 0)),
            pl.BlockSpec((1, 1, 2 * HID), lambda t, g, vv: (g[t], 0, 0)),
            pl.BlockSpec((1, HID, D), lambda t, g, vv: (g[t], 0, 0)),
            pl.BlockSpec((1, 1, D), lambda t, g, vv: (g[t], 0, 0)),
        ],
        out_specs=[pl.BlockSpec((TILE, QC), lambda t, g, vv: (t, 0))
                   for _ in range(4)],
    )
    return pl.pallas_call(
        _moe_body,
        grid_spec=grid_spec,
        out_shape=[jax.ShapeDtypeStruct((PADDED, QC), f32) for _ in range(4)],
    )(gid, valid, hq[0], hq[1], hq[2], hq[3], w1b, b1, w2b, b2)


def _addres_body(a_ref, m0_ref, m1_ref, m2_ref, m3_ref, o_ref):
    a = a_ref[...]
    mrefs = (m0_ref, m1_ref, m2_ref, m3_ref)
    for c in range(4):
        o_ref[:, c * QC:(c + 1) * QC] = a[:, c * QC:(c + 1) * QC] + mrefs[c][...]


def _add_residual(a, mq):
    nsb = S // SB
    return pl.pallas_call(
        _addres_body,
        grid=(nsb,),
        in_specs=[pl.BlockSpec((SB, D), lambda i: (i, 0))] +
                 [pl.BlockSpec((SB, QC), lambda i: (i, 0)) for _ in range(4)],
        out_specs=pl.BlockSpec((SB, D), lambda i: (i, 0)),
        out_shape=jax.ShapeDtypeStruct((S, D), f32),
    )(a, mq[0], mq[1], mq[2], mq[3])


def kernel(x, ln1_w, Wq, bq, Wk, bk, Wv, bv, Wg, bg, Wo, bo, qn_w, kn_w,
           ln2_w, Wr, br, W1, b1, W2, b2, cos, sin):
    x2d = x.reshape(S, D)

    q, k, v = _preattn(x2d, ln1_w.reshape(1, D), Wq.astype(bf16),
                       bq.reshape(1, D), Wk.astype(bf16), bk.reshape(1, KV * HD),
                       Wv.astype(bf16), bv.reshape(1, KV * HD),
                       cos, sin, qn_w.reshape(1, HD), kn_w.reshape(1, HD))
    ri = lax.broadcasted_iota(jnp.int32, (AQ, AKV), 0)
    ci = lax.broadcasted_iota(jnp.int32, (AQ, AKV), 1)
    nega = jnp.where(ri >= ci, 0.0, -1e30).astype(f32)
    negb = jnp.where(ri >= ci + AKV, 0.0, -1e30).astype(f32)
    attn = _attention(q, k, v, Wg.astype(bf16), bg.reshape(1, HD), nega, negb)
    x2, oneh, *h2q = _post_router(attn, x2d, Wo.astype(bf16), bo.reshape(1, D),
                                  ln2_w.reshape(1, D), Wr.astype(bf16),
                                  br.reshape(1, E))
    dest, gid, valid = _routing(oneh)
    dest_row = dest.reshape(1, S)
    hs_q = [_sc_scatter_rows(h2q[c], dest_row) for c in range(4)]
    yq = _moe(gid.reshape(NTILES), valid.reshape(NTILES), hs_q,
              W1, b1.reshape(E, 1, 2 * HID),
              W2, b2.reshape(E, 1, D))
    mq = [_sc_gather_rows(yq[c], dest_row) for c in range(4)]
    out = _add_residual(x2, mq)
    return out.reshape(B, S, D)
